# baseline (speedup 1.0000x reference)
import jax
import jax.numpy as jnp
from jax import lax
from jax.experimental import pallas as pl
from jax.experimental.pallas import tpu as pltpu
from jax.experimental.pallas import tpu_sc as plsc

N = 10000
E = 320000
D = 128
NC = 2
NS = 16
NW = NC * NS
N_P = NS * 640
RPS = N_P // NS
K = 128
NCH = 80
EPW = NCH * K
E_P = NW * EPW
LANES = 16
BLK = 512
GRID = N_P // BLK


def _sc_mesh():
    return plsc.VectorSubcoreMesh(
        core_axis_name="c", subcore_axis_name="s", num_cores=NC, num_subcores=NS
    )


def _deg_body(src_hbm, dst_hbm, out_hbm, sidx, didx, ones, zb, ho, hi, sem0, sem1):
    c = lax.axis_index("c")
    s = lax.axis_index("s")
    w = c * NS + s
    pltpu.sync_copy(src_hbm.at[w], sidx)
    pltpu.sync_copy(dst_hbm.at[w], didx)
    zero = jnp.zeros((LANES,), jnp.float32)
    one = jnp.ones((LANES,), jnp.float32)
    for j in range(K // LANES):
        ones[pl.ds(j * LANES, LANES)] = one

    def zfill(i, _):
        zb[pl.ds(i * LANES, LANES)] = zero
        return ()

    lax.fori_loop(0, RPS // LANES, zfill, ())
    pltpu.sync_copy(zb, ho.at[pl.ds(s * RPS, RPS)])
    pltpu.sync_copy(zb, hi.at[pl.ds(s * RPS, RPS)])
    plsc.subcore_barrier()

    def fire(i, _):
        pltpu.async_copy(ones, ho.at[sidx.at[i]], sem0, add=True)
        pltpu.async_copy(ones, hi.at[didx.at[i]], sem1, add=True)
        return ()

    lax.fori_loop(0, NCH, fire, ())

    def drain(i, _):
        pltpu.make_async_copy(ones, ho.at[sidx.at[i]], sem0).wait()
        pltpu.make_async_copy(ones, hi.at[didx.at[i]], sem1).wait()
        return ()

    lax.fori_loop(0, NCH, drain, ())
    plsc.subcore_barrier()
    pltpu.sync_copy(ho.at[pl.ds(s * RPS, RPS)], out_hbm.at[c, 0, pl.ds(s * RPS, RPS)])
    pltpu.sync_copy(hi.at[pl.ds(s * RPS, RPS)], out_hbm.at[c, 1, pl.ds(s * RPS, RPS)])


def _deg(src3, dst3):
    fn = pl.kernel(
        _deg_body,
        out_type=jax.ShapeDtypeStruct((NC, 2, N_P), jnp.float32),
        mesh=_sc_mesh(),
        scratch_types=[
            pltpu.VMEM((NCH, K), jnp.int32),
            pltpu.VMEM((NCH, K), jnp.int32),
            pltpu.VMEM((K,), jnp.float32),
            pltpu.VMEM((RPS,), jnp.float32),
            pltpu.VMEM_SHARED((N_P,), jnp.float32),
            pltpu.VMEM_SHARED((N_P,), jnp.float32),
            pltpu.SemaphoreType.DMA,
            pltpu.SemaphoreType.DMA,
        ],
    )
    return fn(src3, dst3)


def _spmm(src3, dst3, h):
    def body(
        src_hbm, dst_hbm, h_hbm, out_hbm,
        sidx0, sidx1, didx0, didx1, rows0, rows1, acc,
        semg0, semg1, semi0, semi1,
    ):
        c = lax.axis_index("c")
        s = lax.axis_index("s")
        w = c * NS + s
        pltpu.sync_copy(src_hbm.at[w, 0], sidx0)
        pltpu.sync_copy(dst_hbm.at[w, 0], didx0)

        zero = jnp.zeros((LANES,), jnp.float32)

        def zfill(i, _):
            for j in range(D // LANES):
                rows0[i, pl.ds(j * LANES, LANES)] = zero
            return ()

        lax.fori_loop(0, K, zfill, ())
        for t in range(RPS // K):
            pltpu.sync_copy(rows0, acc.at[pl.ds(s * RPS + t * K, K)])
        pltpu.async_copy(h_hbm.at[sidx0], rows0, semg0)
        pltpu.async_copy(src_hbm.at[w, 1], sidx1, semi1)
        pltpu.async_copy(dst_hbm.at[w, 1], didx1, semi1)
        plsc.subcore_barrier()

        def window(i, sb, db, rb, semg_b, semi_b, snb, dnb, rnb, semg_nb, semi_nb,
                   has_next, has_next2):
            pltpu.make_async_copy(h_hbm.at[sb], rb, semg_b).wait()
            if has_next is True:
                pltpu.make_async_copy(src_hbm.at[w, i + 1], snb, semi_nb).wait()
                pltpu.make_async_copy(dst_hbm.at[w, i + 1], dnb, semi_nb).wait()
                pltpu.async_copy(h_hbm.at[snb], rnb, semg_nb)
            else:
                @pl.when(has_next)
                def _():
                    pltpu.make_async_copy(src_hbm.at[w, i + 1], snb, semi_nb).wait()
                    pltpu.make_async_copy(dst_hbm.at[w, i + 1], dnb, semi_nb).wait()
                    pltpu.async_copy(h_hbm.at[snb], rnb, semg_nb)

            @pl.when(has_next2)
            def _():
                pltpu.async_copy(src_hbm.at[w, i + 2], sb, semi_b)

            pltpu.sync_copy(rb, acc.at[db], add=True)

            @pl.when(has_next2)
            def _():
                pltpu.async_copy(dst_hbm.at[w, i + 2], db, semi_b)

        def dbl(it, _):
            i0 = 2 * it
            i1 = i0 + 1
            not_last = it < NCH // 2 - 1
            window(i0, sidx0, didx0, rows0, semg0, semi0,
                   sidx1, didx1, rows1, semg1, semi1, True, not_last)
            window(i1, sidx1, didx1, rows1, semg1, semi1,
                   sidx0, didx0, rows0, semg0, semi0, not_last, not_last)
            return ()

        lax.fori_loop(0, NCH // 2, dbl, ())
        plsc.subcore_barrier()
        pltpu.sync_copy(
            acc.at[pl.ds(s * RPS, RPS)], out_hbm.at[c, pl.ds(s * RPS, RPS)]
        )

    fn = pl.kernel(
        body,
        out_type=jax.ShapeDtypeStruct((NC, N_P, D), jnp.float32),
        mesh=_sc_mesh(),
        scratch_types=[
            pltpu.VMEM((K,), jnp.int32),
            pltpu.VMEM((K,), jnp.int32),
            pltpu.VMEM((K,), jnp.int32),
            pltpu.VMEM((K,), jnp.int32),
            pltpu.VMEM((K, D), jnp.float32),
            pltpu.VMEM((K, D), jnp.float32),
            pltpu.VMEM_SHARED((N_P, D), jnp.float32),
            pltpu.SemaphoreType.DMA,
            pltpu.SemaphoreType.DMA,
            pltpu.SemaphoreType.DMA,
            pltpu.SemaphoreType.DMA,
        ],
    )
    return fn(src3, dst3, h)


def _prep_body(degp_ref, x_ref, hs_ref, ns_ref, nd_ref):
    dp = degp_ref[...]
    deg_out = dp[0, 0] + dp[1, 0]
    deg_in = dp[0, 1] + dp[1, 1]
    ns = jnp.where(deg_out > 0, lax.rsqrt(jnp.maximum(deg_out, 1e-12)), 0.0)
    nd = jnp.where(deg_in > 0, lax.rsqrt(jnp.maximum(deg_in, 1e-12)), 0.0)
    ns_ref[...] = ns[:, None]
    nd_ref[...] = nd[:, None]
    hs_ref[...] = x_ref[...] * ns[:, None]


def _prep(degp, xp):
    return pl.pallas_call(
        _prep_body,
        grid=(GRID,),
        in_specs=[
            pl.BlockSpec((NC, 2, BLK), lambda i: (0, 0, i)),
            pl.BlockSpec((BLK, D), lambda i: (i, 0)),
        ],
        out_specs=[
            pl.BlockSpec((BLK, D), lambda i: (i, 0)),
            pl.BlockSpec((BLK, 1), lambda i: (i, 0)),
            pl.BlockSpec((BLK, 1), lambda i: (i, 0)),
        ],
        out_shape=[
            jax.ShapeDtypeStruct((N_P, D), jnp.float32),
            jax.ShapeDtypeStruct((N_P, 1), jnp.float32),
            jax.ShapeDtypeStruct((N_P, 1), jnp.float32),
        ],
    )(degp, xp)


def _layer_body(ap_ref, nd_ref, ns_ref, w_ref, b_ref, h_ref, hs_ref):
    p = (ap_ref[0] + ap_ref[1]) * nd_ref[...]
    z = jnp.dot(p, w_ref[...], preferred_element_type=jnp.float32) + b_ref[...]
    h = jnp.maximum(z, 0.0)
    h_ref[...] = h
    hs_ref[...] = h * ns_ref[...]


def _layer_s_body(ap_ref, nd_ref, ns_ref, w_ref, b_ref, hs_ref):
    p = (ap_ref[0] + ap_ref[1]) * nd_ref[...]
    z = jnp.dot(p, w_ref[...], preferred_element_type=jnp.float32) + b_ref[...]
    hs_ref[...] = jnp.maximum(z, 0.0) * ns_ref[...]


def _layer(ap, nd, ns, w, b, scaled_only=False):
    in_specs = [
        pl.BlockSpec((NC, BLK, D), lambda i: (0, i, 0)),
        pl.BlockSpec((BLK, 1), lambda i: (i, 0)),
        pl.BlockSpec((BLK, 1), lambda i: (i, 0)),
        pl.BlockSpec((D, D), lambda i: (0, 0)),
        pl.BlockSpec((1, D), lambda i: (0, 0)),
    ]
    out_spec = pl.BlockSpec((BLK, D), lambda i: (i, 0))
    out_shape = jax.ShapeDtypeStruct((N_P, D), jnp.float32)
    if scaled_only:
        return pl.pallas_call(
            _layer_s_body,
            grid=(GRID,),
            in_specs=in_specs,
            out_specs=[out_spec],
            out_shape=[out_shape],
        )(ap, nd, ns, w, b)[0]
    return pl.pallas_call(
        _layer_body,
        grid=(GRID,),
        in_specs=in_specs,
        out_specs=[out_spec, out_spec],
        out_shape=[out_shape, out_shape],
    )(ap, nd, ns, w, b)


def _final_body(ap_ref, nd_ref, w2_ref, b2_ref, w3_ref, b3_ref, h3_ref, h4_ref):
    p = (ap_ref[0] + ap_ref[1]) * nd_ref[...]
    z3 = jnp.dot(p, w2_ref[...], preferred_element_type=jnp.float32) + b2_ref[...]
    h3_ref[...] = jnp.maximum(z3, 0.0)
    h4_ref[...] = (
        jnp.dot(p, w3_ref[...], preferred_element_type=jnp.float32) + b3_ref[...]
    )


def _final(ap, nd, w2, b2, w3, b3):
    blk = 400
    return pl.pallas_call(
        _final_body,
        grid=(N // blk,),
        in_specs=[
            pl.BlockSpec((NC, blk, D), lambda i: (0, i, 0)),
            pl.BlockSpec((blk, 1), lambda i: (i, 0)),
            pl.BlockSpec((D, D), lambda i: (0, 0)),
            pl.BlockSpec((1, D), lambda i: (0, 0)),
            pl.BlockSpec((D, D), lambda i: (0, 0)),
            pl.BlockSpec((1, D), lambda i: (0, 0)),
        ],
        out_specs=[
            pl.BlockSpec((blk, D), lambda i: (i, 0)),
            pl.BlockSpec((blk, D), lambda i: (i, 0)),
        ],
        out_shape=[
            jax.ShapeDtypeStruct((N, D), jnp.float32),
            jax.ShapeDtypeStruct((N, D), jnp.float32),
        ],
    )(ap, nd, w2, b2, w3, b3)


def kernel(edge_index, x, W1, b1, W2, b2, W3, b3):
    src = edge_index[0]
    dst = edge_index[1]
    pad_idx = (N + (jnp.arange(E_P - E, dtype=jnp.int32) % (N_P - N))).astype(
        jnp.int32
    )
    src3 = jnp.concatenate([src, pad_idx]).reshape(NW, NCH, K)
    dst3 = jnp.concatenate([dst, pad_idx]).reshape(NW, NCH, K)
    xp = jnp.pad(x, ((0, N_P - N), (0, 0)))
    b1r = b1.reshape(1, D)
    b2r = b2.reshape(1, D)
    b3r = b3.reshape(1, D)

    degp = _deg(src3, dst3)
    hs0, ns, nd = _prep(degp, xp)
    a1 = _spmm(src3, dst3, hs0)
    h1s = _layer(a1, nd, ns, W1, b1r, scaled_only=True)
    a2 = _spmm(src3, dst3, h1s)
    h2, h2s = _layer(a2, nd, ns, W2, b2r)
    a3 = _spmm(src3, dst3, h2s)
    h3, h4 = _final(a3, nd, W2, b2r, W3, b3r)
    return (h4, h3, h2[:N])


# speedup vs baseline: 1.0378x; 1.0378x over previous
import jax
import jax.numpy as jnp
from jax import lax
from jax.experimental import pallas as pl
from jax.experimental.pallas import tpu as pltpu
from jax.experimental.pallas import tpu_sc as plsc

N = 10000
E = 320000
D = 128
NC = 2
NS = 16
NW = NC * NS
N_P = NS * 640
RPS = N_P // NS
K = 128
NCH = 80
EPW = NCH * K
E_P = NW * EPW
LANES = 16
BLK = 1024
GRID = N_P // BLK


def _sc_mesh():
    return plsc.VectorSubcoreMesh(
        core_axis_name="c", subcore_axis_name="s", num_cores=NC, num_subcores=NS
    )


def _deg_body(src_hbm, dst_hbm, out_hbm, sidx, didx, ones, zb, ho, hi, sem0, sem1):
    c = lax.axis_index("c")
    s = lax.axis_index("s")
    w = c * NS + s
    pltpu.sync_copy(src_hbm.at[w], sidx)
    pltpu.sync_copy(dst_hbm.at[w], didx)
    zero = jnp.zeros((LANES,), jnp.float32)
    one = jnp.ones((LANES,), jnp.float32)
    for j in range(K // LANES):
        ones[pl.ds(j * LANES, LANES)] = one

    def zfill(i, _):
        zb[pl.ds(i * LANES, LANES)] = zero
        return ()

    lax.fori_loop(0, RPS // LANES, zfill, ())
    pltpu.sync_copy(zb, ho.at[pl.ds(s * RPS, RPS)])
    pltpu.sync_copy(zb, hi.at[pl.ds(s * RPS, RPS)])
    plsc.subcore_barrier()

    def fire(i, _):
        pltpu.async_copy(ones, ho.at[sidx.at[i]], sem0, add=True)
        pltpu.async_copy(ones, hi.at[didx.at[i]], sem1, add=True)
        return ()

    lax.fori_loop(0, NCH, fire, ())

    def drain(i, _):
        pltpu.make_async_copy(ones, ho.at[sidx.at[i]], sem0).wait()
        pltpu.make_async_copy(ones, hi.at[didx.at[i]], sem1).wait()
        return ()

    lax.fori_loop(0, NCH, drain, ())
    plsc.subcore_barrier()
    pltpu.sync_copy(ho.at[pl.ds(s * RPS, RPS)], out_hbm.at[c, 0, pl.ds(s * RPS, RPS)])
    pltpu.sync_copy(hi.at[pl.ds(s * RPS, RPS)], out_hbm.at[c, 1, pl.ds(s * RPS, RPS)])


def _deg(src3, dst3):
    fn = pl.kernel(
        _deg_body,
        out_type=jax.ShapeDtypeStruct((NC, 2, N_P), jnp.float32),
        mesh=_sc_mesh(),
        scratch_types=[
            pltpu.VMEM((NCH, K), jnp.int32),
            pltpu.VMEM((NCH, K), jnp.int32),
            pltpu.VMEM((K,), jnp.float32),
            pltpu.VMEM((RPS,), jnp.float32),
            pltpu.VMEM_SHARED((N_P,), jnp.float32),
            pltpu.VMEM_SHARED((N_P,), jnp.float32),
            pltpu.SemaphoreType.DMA,
            pltpu.SemaphoreType.DMA,
        ],
    )
    return fn(src3, dst3)


def _spmm(src3, dst3, h):
    def body(
        src_hbm, dst_hbm, h_hbm, out_hbm,
        sidx0, sidx1, didx0, didx1, rows0, rows1, acc,
        semg0, semg1, semi0, semi1,
    ):
        c = lax.axis_index("c")
        s = lax.axis_index("s")
        w = c * NS + s
        pltpu.sync_copy(src_hbm.at[w, 0], sidx0)
        pltpu.sync_copy(dst_hbm.at[w, 0], didx0)
        pltpu.async_copy(h_hbm.at[sidx0], rows1, semg0)
        pltpu.async_copy(src_hbm.at[w, 1], sidx1, semi1)
        pltpu.async_copy(dst_hbm.at[w, 1], didx1, semi1)

        zero = jnp.zeros((LANES,), jnp.float32)

        def zfill(i, _):
            for j in range(D // LANES):
                rows0[i, pl.ds(j * LANES, LANES)] = zero
            return ()

        lax.fori_loop(0, K, zfill, ())
        for t in range(RPS // K):
            pltpu.sync_copy(rows0, acc.at[pl.ds(s * RPS + t * K, K)])
        plsc.subcore_barrier()

        def window(i, sb, db, rb, semg_b, semi_b, snb, dnb, rnb, semg_nb, semi_nb,
                   has_next, has_next2):
            pltpu.make_async_copy(h_hbm.at[sb], rb, semg_b).wait()
            if has_next is True:
                pltpu.make_async_copy(src_hbm.at[w, i + 1], snb, semi_nb).wait()
                pltpu.make_async_copy(dst_hbm.at[w, i + 1], dnb, semi_nb).wait()
                pltpu.async_copy(h_hbm.at[snb], rnb, semg_nb)
            else:
                @pl.when(has_next)
                def _():
                    pltpu.make_async_copy(src_hbm.at[w, i + 1], snb, semi_nb).wait()
                    pltpu.make_async_copy(dst_hbm.at[w, i + 1], dnb, semi_nb).wait()
                    pltpu.async_copy(h_hbm.at[snb], rnb, semg_nb)

            @pl.when(has_next2)
            def _():
                pltpu.async_copy(src_hbm.at[w, i + 2], sb, semi_b)

            pltpu.sync_copy(rb, acc.at[db], add=True)

            @pl.when(has_next2)
            def _():
                pltpu.async_copy(dst_hbm.at[w, i + 2], db, semi_b)

        def dbl(it, _):
            i0 = 2 * it
            i1 = i0 + 1
            not_last = it < NCH // 2 - 1
            window(i0, sidx0, didx0, rows1, semg0, semi0,
                   sidx1, didx1, rows0, semg1, semi1, True, not_last)
            window(i1, sidx1, didx1, rows0, semg1, semi1,
                   sidx0, didx0, rows1, semg0, semi0, not_last, not_last)
            return ()

        lax.fori_loop(0, NCH // 2, dbl, ())
        plsc.subcore_barrier()
        pltpu.sync_copy(
            acc.at[pl.ds(s * RPS, RPS)], out_hbm.at[c, pl.ds(s * RPS, RPS)]
        )

    fn = pl.kernel(
        body,
        out_type=jax.ShapeDtypeStruct((NC, N_P, D), jnp.float32),
        mesh=_sc_mesh(),
        scratch_types=[
            pltpu.VMEM((K,), jnp.int32),
            pltpu.VMEM((K,), jnp.int32),
            pltpu.VMEM((K,), jnp.int32),
            pltpu.VMEM((K,), jnp.int32),
            pltpu.VMEM((K, D), jnp.float32),
            pltpu.VMEM((K, D), jnp.float32),
            pltpu.VMEM_SHARED((N_P, D), jnp.float32),
            pltpu.SemaphoreType.DMA,
            pltpu.SemaphoreType.DMA,
            pltpu.SemaphoreType.DMA,
            pltpu.SemaphoreType.DMA,
        ],
    )
    return fn(src3, dst3, h)


def _prep_body(degp_ref, x_ref, hs_ref, ns_ref, nd_ref):
    dp = degp_ref[...]
    deg_out = dp[0, 0] + dp[1, 0]
    deg_in = dp[0, 1] + dp[1, 1]
    ns = jnp.where(deg_out > 0, lax.rsqrt(jnp.maximum(deg_out, 1e-12)), 0.0)
    nd = jnp.where(deg_in > 0, lax.rsqrt(jnp.maximum(deg_in, 1e-12)), 0.0)
    ns_ref[...] = ns[:, None]
    nd_ref[...] = nd[:, None]
    hs_ref[...] = x_ref[...] * ns[:, None]


def _prep(degp, xp):
    return pl.pallas_call(
        _prep_body,
        grid=(GRID,),
        in_specs=[
            pl.BlockSpec((NC, 2, BLK), lambda i: (0, 0, i)),
            pl.BlockSpec((BLK, D), lambda i: (i, 0)),
        ],
        out_specs=[
            pl.BlockSpec((BLK, D), lambda i: (i, 0)),
            pl.BlockSpec((BLK, 1), lambda i: (i, 0)),
            pl.BlockSpec((BLK, 1), lambda i: (i, 0)),
        ],
        out_shape=[
            jax.ShapeDtypeStruct((N_P, D), jnp.float32),
            jax.ShapeDtypeStruct((N_P, 1), jnp.float32),
            jax.ShapeDtypeStruct((N_P, 1), jnp.float32),
        ],
    )(degp, xp)


def _layer_body(ap_ref, nd_ref, ns_ref, w_ref, b_ref, h_ref, hs_ref):
    p = (ap_ref[0] + ap_ref[1]) * nd_ref[...]
    z = jnp.dot(p, w_ref[...], preferred_element_type=jnp.float32) + b_ref[...]
    h = jnp.maximum(z, 0.0)
    h_ref[...] = h
    hs_ref[...] = h * ns_ref[...]


def _layer_s_body(ap_ref, nd_ref, ns_ref, w_ref, b_ref, hs_ref):
    p = (ap_ref[0] + ap_ref[1]) * nd_ref[...]
    z = jnp.dot(p, w_ref[...], preferred_element_type=jnp.float32) + b_ref[...]
    hs_ref[...] = jnp.maximum(z, 0.0) * ns_ref[...]


def _layer(ap, nd, ns, w, b, scaled_only=False):
    in_specs = [
        pl.BlockSpec((NC, BLK, D), lambda i: (0, i, 0)),
        pl.BlockSpec((BLK, 1), lambda i: (i, 0)),
        pl.BlockSpec((BLK, 1), lambda i: (i, 0)),
        pl.BlockSpec((D, D), lambda i: (0, 0)),
        pl.BlockSpec((1, D), lambda i: (0, 0)),
    ]
    out_spec = pl.BlockSpec((BLK, D), lambda i: (i, 0))
    out_shape = jax.ShapeDtypeStruct((N_P, D), jnp.float32)
    if scaled_only:
        return pl.pallas_call(
            _layer_s_body,
            grid=(GRID,),
            in_specs=in_specs,
            out_specs=[out_spec],
            out_shape=[out_shape],
        )(ap, nd, ns, w, b)[0]
    return pl.pallas_call(
        _layer_body,
        grid=(GRID,),
        in_specs=in_specs,
        out_specs=[out_spec, out_spec],
        out_shape=[out_shape, out_shape],
    )(ap, nd, ns, w, b)


def _final_body(ap_ref, nd_ref, w2_ref, b2_ref, w3_ref, b3_ref, h3_ref, h4_ref):
    p = (ap_ref[0] + ap_ref[1]) * nd_ref[...]
    z3 = jnp.dot(p, w2_ref[...], preferred_element_type=jnp.float32) + b2_ref[...]
    h3_ref[...] = jnp.maximum(z3, 0.0)
    h4_ref[...] = (
        jnp.dot(p, w3_ref[...], preferred_element_type=jnp.float32) + b3_ref[...]
    )


def _final(ap, nd, w2, b2, w3, b3):
    blk = 400
    return pl.pallas_call(
        _final_body,
        grid=(N // blk,),
        in_specs=[
            pl.BlockSpec((NC, blk, D), lambda i: (0, i, 0)),
            pl.BlockSpec((blk, 1), lambda i: (i, 0)),
            pl.BlockSpec((D, D), lambda i: (0, 0)),
            pl.BlockSpec((1, D), lambda i: (0, 0)),
            pl.BlockSpec((D, D), lambda i: (0, 0)),
            pl.BlockSpec((1, D), lambda i: (0, 0)),
        ],
        out_specs=[
            pl.BlockSpec((blk, D), lambda i: (i, 0)),
            pl.BlockSpec((blk, D), lambda i: (i, 0)),
        ],
        out_shape=[
            jax.ShapeDtypeStruct((N, D), jnp.float32),
            jax.ShapeDtypeStruct((N, D), jnp.float32),
        ],
    )(ap, nd, w2, b2, w3, b3)


def kernel(edge_index, x, W1, b1, W2, b2, W3, b3):
    src = edge_index[0]
    dst = edge_index[1]
    pad_idx = (N + (jnp.arange(E_P - E, dtype=jnp.int32) % (N_P - N))).astype(
        jnp.int32
    )
    src3 = jnp.concatenate([src, pad_idx]).reshape(NW, NCH, K)
    dst3 = jnp.concatenate([dst, pad_idx]).reshape(NW, NCH, K)
    xp = jnp.pad(x, ((0, N_P - N), (0, 0)))
    b1r = b1.reshape(1, D)
    b2r = b2.reshape(1, D)
    b3r = b3.reshape(1, D)

    degp = _deg(src3, dst3)
    hs0, ns, nd = _prep(degp, xp)
    a1 = _spmm(src3, dst3, hs0)
    h1s = _layer(a1, nd, ns, W1, b1r, scaled_only=True)
    a2 = _spmm(src3, dst3, h1s)
    h2, h2s = _layer(a2, nd, ns, W2, b2r)
    a3 = _spmm(src3, dst3, h2s)
    h3, h4 = _final(a3, nd, W2, b2r, W3, b3r)
    return (h4, h3, h2[:N])


# speedup vs baseline: 1.0441x; 1.0060x over previous
import jax
import jax.numpy as jnp
from jax import lax
from jax.experimental import pallas as pl
from jax.experimental.pallas import tpu as pltpu
from jax.experimental.pallas import tpu_sc as plsc

N = 10000
E = 320000
D = 128
NC = 2
NS = 16
NW = NC * NS
N_P = NS * 640
RPS = N_P // NS
K = 128
NCH = 80
EPW = NCH * K
E_P = NW * EPW
LANES = 16
BLK = 1024
GRID = N_P // BLK


def _sc_mesh():
    return plsc.VectorSubcoreMesh(
        core_axis_name="c", subcore_axis_name="s", num_cores=NC, num_subcores=NS
    )


def _deg_body(src_hbm, dst_hbm, out_hbm, sidx, didx, ones, zb, ho, hi, sem0, sem1):
    c = lax.axis_index("c")
    s = lax.axis_index("s")
    w = c * NS + s
    pltpu.sync_copy(src_hbm.at[w], sidx)
    pltpu.sync_copy(dst_hbm.at[w], didx)
    zero = jnp.zeros((LANES,), jnp.float32)
    one = jnp.ones((LANES,), jnp.float32)
    for j in range(K // LANES):
        ones[pl.ds(j * LANES, LANES)] = one

    def zfill(i, _):
        zb[pl.ds(i * LANES, LANES)] = zero
        return ()

    lax.fori_loop(0, RPS // LANES, zfill, ())
    pltpu.sync_copy(zb, ho.at[pl.ds(s * RPS, RPS)])
    pltpu.sync_copy(zb, hi.at[pl.ds(s * RPS, RPS)])
    plsc.subcore_barrier()

    def fire(i, _):
        pltpu.async_copy(ones, ho.at[sidx.at[i]], sem0, add=True)
        pltpu.async_copy(ones, hi.at[didx.at[i]], sem1, add=True)
        return ()

    lax.fori_loop(0, NCH, fire, ())

    def drain(i, _):
        pltpu.make_async_copy(ones, ho.at[sidx.at[i]], sem0).wait()
        pltpu.make_async_copy(ones, hi.at[didx.at[i]], sem1).wait()
        return ()

    lax.fori_loop(0, NCH, drain, ())
    plsc.subcore_barrier()
    pltpu.sync_copy(ho.at[pl.ds(s * RPS, RPS)], out_hbm.at[c, 0, pl.ds(s * RPS, RPS)])
    pltpu.sync_copy(hi.at[pl.ds(s * RPS, RPS)], out_hbm.at[c, 1, pl.ds(s * RPS, RPS)])


def _deg(src3, dst3):
    fn = pl.kernel(
        _deg_body,
        out_type=jax.ShapeDtypeStruct((NC, 2, N_P), jnp.float32),
        mesh=_sc_mesh(),
        scratch_types=[
            pltpu.VMEM((NCH, K), jnp.int32),
            pltpu.VMEM((NCH, K), jnp.int32),
            pltpu.VMEM((K,), jnp.float32),
            pltpu.VMEM((RPS,), jnp.float32),
            pltpu.VMEM_SHARED((N_P,), jnp.float32),
            pltpu.VMEM_SHARED((N_P,), jnp.float32),
            pltpu.SemaphoreType.DMA,
            pltpu.SemaphoreType.DMA,
        ],
    )
    return fn(src3, dst3)


def _spmm(src3, dst3, h):
    def body(
        src_hbm, dst_hbm, h_hbm, out_hbm,
        sidx0, sidx1, didx0, didx1, rows0, rows1, acc,
        semg0, semg1, semi0, semi1,
    ):
        c = lax.axis_index("c")
        s = lax.axis_index("s")
        w = c * NS + s
        pltpu.sync_copy(src_hbm.at[w, 0], sidx0)
        pltpu.sync_copy(dst_hbm.at[w, 0], didx0)
        pltpu.async_copy(h_hbm.at[sidx0], rows1, semg0)
        pltpu.async_copy(src_hbm.at[w, 1], sidx1, semi1)
        pltpu.async_copy(dst_hbm.at[w, 1], didx1, semi1)

        zero = jnp.zeros((LANES,), jnp.float32)

        def zfill(i, _):
            for j in range(D // LANES):
                rows0[i, pl.ds(j * LANES, LANES)] = zero
            return ()

        lax.fori_loop(0, K, zfill, ())
        for t in range(RPS // K):
            pltpu.sync_copy(rows0, acc.at[pl.ds(s * RPS + t * K, K)])
        plsc.subcore_barrier()

        def window(i, sb, db, rb, semg_b, semi_b, snb, dnb, rnb, semg_nb, semi_nb,
                   has_next, has_next2):
            if has_next is True:
                pltpu.make_async_copy(src_hbm.at[w, i + 1], snb, semi_nb).wait()
                pltpu.make_async_copy(dst_hbm.at[w, i + 1], dnb, semi_nb).wait()
                pltpu.make_async_copy(h_hbm.at[sb], rb, semg_b).wait()
                pltpu.async_copy(h_hbm.at[snb], rnb, semg_nb)
            else:
                @pl.when(has_next)
                def _():
                    pltpu.make_async_copy(src_hbm.at[w, i + 1], snb, semi_nb).wait()
                    pltpu.make_async_copy(dst_hbm.at[w, i + 1], dnb, semi_nb).wait()

                pltpu.make_async_copy(h_hbm.at[sb], rb, semg_b).wait()

                @pl.when(has_next)
                def _():
                    pltpu.async_copy(h_hbm.at[snb], rnb, semg_nb)

            @pl.when(has_next2)
            def _():
                pltpu.async_copy(src_hbm.at[w, i + 2], sb, semi_b)

            pltpu.sync_copy(rb, acc.at[db], add=True)

            @pl.when(has_next2)
            def _():
                pltpu.async_copy(dst_hbm.at[w, i + 2], db, semi_b)

        def dbl(it, _):
            i0 = 2 * it
            i1 = i0 + 1
            not_last = it < NCH // 2 - 1
            window(i0, sidx0, didx0, rows1, semg0, semi0,
                   sidx1, didx1, rows0, semg1, semi1, True, not_last)
            window(i1, sidx1, didx1, rows0, semg1, semi1,
                   sidx0, didx0, rows1, semg0, semi0, not_last, not_last)
            return ()

        lax.fori_loop(0, NCH // 2, dbl, ())
        plsc.subcore_barrier()
        pltpu.sync_copy(
            acc.at[pl.ds(s * RPS, RPS)], out_hbm.at[c, pl.ds(s * RPS, RPS)]
        )

    fn = pl.kernel(
        body,
        out_type=jax.ShapeDtypeStruct((NC, N_P, D), jnp.float32),
        mesh=_sc_mesh(),
        scratch_types=[
            pltpu.VMEM((K,), jnp.int32),
            pltpu.VMEM((K,), jnp.int32),
            pltpu.VMEM((K,), jnp.int32),
            pltpu.VMEM((K,), jnp.int32),
            pltpu.VMEM((K, D), jnp.float32),
            pltpu.VMEM((K, D), jnp.float32),
            pltpu.VMEM_SHARED((N_P, D), jnp.float32),
            pltpu.SemaphoreType.DMA,
            pltpu.SemaphoreType.DMA,
            pltpu.SemaphoreType.DMA,
            pltpu.SemaphoreType.DMA,
        ],
    )
    return fn(src3, dst3, h)


def _prep_body(degp_ref, x_ref, hs_ref, ns_ref, nd_ref):
    dp = degp_ref[...]
    deg_out = dp[0, 0] + dp[1, 0]
    deg_in = dp[0, 1] + dp[1, 1]
    ns = jnp.where(deg_out > 0, lax.rsqrt(jnp.maximum(deg_out, 1e-12)), 0.0)
    nd = jnp.where(deg_in > 0, lax.rsqrt(jnp.maximum(deg_in, 1e-12)), 0.0)
    ns_ref[...] = ns[:, None]
    nd_ref[...] = nd[:, None]
    hs_ref[...] = x_ref[...] * ns[:, None]


def _prep(degp, xp):
    return pl.pallas_call(
        _prep_body,
        grid=(GRID,),
        in_specs=[
            pl.BlockSpec((NC, 2, BLK), lambda i: (0, 0, i)),
            pl.BlockSpec((BLK, D), lambda i: (i, 0)),
        ],
        out_specs=[
            pl.BlockSpec((BLK, D), lambda i: (i, 0)),
            pl.BlockSpec((BLK, 1), lambda i: (i, 0)),
            pl.BlockSpec((BLK, 1), lambda i: (i, 0)),
        ],
        out_shape=[
            jax.ShapeDtypeStruct((N_P, D), jnp.float32),
            jax.ShapeDtypeStruct((N_P, 1), jnp.float32),
            jax.ShapeDtypeStruct((N_P, 1), jnp.float32),
        ],
    )(degp, xp)


def _layer_body(ap_ref, nd_ref, ns_ref, w_ref, b_ref, h_ref, hs_ref):
    p = (ap_ref[0] + ap_ref[1]) * nd_ref[...]
    z = jnp.dot(p, w_ref[...], preferred_element_type=jnp.float32) + b_ref[...]
    h = jnp.maximum(z, 0.0)
    h_ref[...] = h
    hs_ref[...] = h * ns_ref[...]


def _layer_s_body(ap_ref, nd_ref, ns_ref, w_ref, b_ref, hs_ref):
    p = (ap_ref[0] + ap_ref[1]) * nd_ref[...]
    z = jnp.dot(p, w_ref[...], preferred_element_type=jnp.float32) + b_ref[...]
    hs_ref[...] = jnp.maximum(z, 0.0) * ns_ref[...]


def _layer(ap, nd, ns, w, b, scaled_only=False):
    in_specs = [
        pl.BlockSpec((NC, BLK, D), lambda i: (0, i, 0)),
        pl.BlockSpec((BLK, 1), lambda i: (i, 0)),
        pl.BlockSpec((BLK, 1), lambda i: (i, 0)),
        pl.BlockSpec((D, D), lambda i: (0, 0)),
        pl.BlockSpec((1, D), lambda i: (0, 0)),
    ]
    out_spec = pl.BlockSpec((BLK, D), lambda i: (i, 0))
    out_shape = jax.ShapeDtypeStruct((N_P, D), jnp.float32)
    if scaled_only:
        return pl.pallas_call(
            _layer_s_body,
            grid=(GRID,),
            in_specs=in_specs,
            out_specs=[out_spec],
            out_shape=[out_shape],
        )(ap, nd, ns, w, b)[0]
    return pl.pallas_call(
        _layer_body,
        grid=(GRID,),
        in_specs=in_specs,
        out_specs=[out_spec, out_spec],
        out_shape=[out_shape, out_shape],
    )(ap, nd, ns, w, b)


def _final_body(ap_ref, nd_ref, w2_ref, b2_ref, w3_ref, b3_ref, h3_ref, h4_ref):
    p = (ap_ref[0] + ap_ref[1]) * nd_ref[...]
    z3 = jnp.dot(p, w2_ref[...], preferred_element_type=jnp.float32) + b2_ref[...]
    h3_ref[...] = jnp.maximum(z3, 0.0)
    h4_ref[...] = (
        jnp.dot(p, w3_ref[...], preferred_element_type=jnp.float32) + b3_ref[...]
    )


def _final(ap, nd, w2, b2, w3, b3):
    blk = 400
    return pl.pallas_call(
        _final_body,
        grid=(N // blk,),
        in_specs=[
            pl.BlockSpec((NC, blk, D), lambda i: (0, i, 0)),
            pl.BlockSpec((blk, 1), lambda i: (i, 0)),
            pl.BlockSpec((D, D), lambda i: (0, 0)),
            pl.BlockSpec((1, D), lambda i: (0, 0)),
            pl.BlockSpec((D, D), lambda i: (0, 0)),
            pl.BlockSpec((1, D), lambda i: (0, 0)),
        ],
        out_specs=[
            pl.BlockSpec((blk, D), lambda i: (i, 0)),
            pl.BlockSpec((blk, D), lambda i: (i, 0)),
        ],
        out_shape=[
            jax.ShapeDtypeStruct((N, D), jnp.float32),
            jax.ShapeDtypeStruct((N, D), jnp.float32),
        ],
    )(ap, nd, w2, b2, w3, b3)


def kernel(edge_index, x, W1, b1, W2, b2, W3, b3):
    src = edge_index[0]
    dst = edge_index[1]
    pad_idx = (N + (jnp.arange(E_P - E, dtype=jnp.int32) % (N_P - N))).astype(
        jnp.int32
    )
    src3 = jnp.concatenate([src, pad_idx]).reshape(NW, NCH, K)
    dst3 = jnp.concatenate([dst, pad_idx]).reshape(NW, NCH, K)
    xp = jnp.pad(x, ((0, N_P - N), (0, 0)))
    b1r = b1.reshape(1, D)
    b2r = b2.reshape(1, D)
    b3r = b3.reshape(1, D)

    degp = _deg(src3, dst3)
    hs0, ns, nd = _prep(degp, xp)
    a1 = _spmm(src3, dst3, hs0)
    h1s = _layer(a1, nd, ns, W1, b1r, scaled_only=True)
    a2 = _spmm(src3, dst3, h1s)
    h2, h2s = _layer(a2, nd, ns, W2, b2r)
    a3 = _spmm(src3, dst3, h2s)
    h3, h4 = _final(a3, nd, W2, b2r, W3, b3r)
    return (h4, h3, h2[:N])
